# baseline (speedup 1.0000x reference)
import functools

import jax
import jax.numpy as jnp
from jax import lax
from jax.experimental import pallas as pl
from jax.experimental.pallas import tpu as pltpu
from jax.experimental.pallas import tpu_sc as plsc

N = 100000
ATOM_TYPES = 119
NUM_GRAPHS = 1024

L = 16
NTILES = 16
CH_MAIN = 6256
CH_TAIL = N - 15 * CH_MAIN
TBL = 128
SEG_PAD = NUM_GRAPHS
SEG_BLK = 128
UNROLL = 4

_mesh = plsc.VectorSubcoreMesh(
    core_axis_name="c", subcore_axis_name="s", num_cores=1)


@functools.partial(
    pl.kernel,
    out_type=jax.ShapeDtypeStruct((N,), jnp.float32),
    mesh=_mesh,
    compiler_params=pltpu.CompilerParams(needs_layout_passes=False),
    scratch_types=[
        pltpu.VMEM((CH_MAIN,), jnp.int32),
        pltpu.VMEM((CH_MAIN,), jnp.int32),
        pltpu.VMEM((CH_MAIN,), jnp.float32),
        pltpu.VMEM((TBL,), jnp.float32),
        pltpu.VMEM((SEG_PAD,), jnp.float32),
        pltpu.VMEM((SEG_PAD,), jnp.float32),
        pltpu.VMEM((NTILES, SEG_BLK), jnp.float32),
        pltpu.VMEM((NTILES, SEG_BLK), jnp.float32),
        pltpu.VMEM((SEG_PAD,), jnp.float32),
        pltpu.VMEM_SHARED((NTILES, SEG_PAD), jnp.float32),
        pltpu.VMEM_SHARED((NTILES, SEG_PAD), jnp.float32),
        pltpu.VMEM_SHARED((SEG_PAD,), jnp.float32),
        pltpu.SemaphoreType.DMA,
        pltpu.SemaphoreType.DMA,
    ],
)
def _sc_multiplier(an_hbm, bid_hbm, fr_hbm, out_hbm,
                   an_v, bid_v, raw_v, tbl_v, acc_sum, acc_cnt,
                   red_sum, red_cnt, inv_v, sh_sum, sh_cnt, sh_inv,
                   sem_a, sem_b):
    w = lax.axis_index("s")
    base = w * CH_MAIN
    nelem = jnp.where(w < 15, CH_MAIN, CH_TAIL)

    with jax.named_scope("ph0_dma_in"):
        cp_a = pltpu.async_copy(an_hbm.at[pl.ds(base, CH_TAIL)],
                                an_v.at[pl.ds(0, CH_TAIL)], sem_a)
        cp_b = pltpu.async_copy(bid_hbm.at[pl.ds(base, CH_TAIL)],
                                bid_v.at[pl.ds(0, CH_TAIL)], sem_b)
        pltpu.sync_copy(fr_hbm, tbl_v)

        eps = jnp.float32(1e-8)
        for j in range(TBL // L):
            f = tbl_v[pl.ds(j * L, L)]
            tbl_v[pl.ds(j * L, L)] = 1.0 / (f + eps)

        zero16 = jnp.zeros((L,), jnp.float32)
        for j in range(SEG_PAD // L):
            acc_sum[pl.ds(j * L, L)] = zero16
            acc_cnt[pl.ds(j * L, L)] = zero16

        @pl.when(w < 15)
        def _():
            pltpu.sync_copy(an_hbm.at[pl.ds(base + CH_TAIL, CH_MAIN - CH_TAIL)],
                            an_v.at[pl.ds(CH_TAIL, CH_MAIN - CH_TAIL)])
            pltpu.sync_copy(bid_hbm.at[pl.ds(base + CH_TAIL, CH_MAIN - CH_TAIL)],
                            bid_v.at[pl.ds(CH_TAIL, CH_MAIN - CH_TAIL)])

        cp_a.wait()
        cp_b.wait()

    ones16 = jnp.ones((L,), jnp.float32)

    with jax.named_scope("ph1_pass1"):
        @plsc.parallel_loop(0, nelem, step=L, unroll=UNROLL)
        def _(i):
            a = an_v[pl.ds(i, L)]
            b = bid_v[pl.ds(i, L)]
            r = plsc.load_gather(tbl_v, [a])
            raw_v[pl.ds(i, L)] = r
            plsc.addupdate_scatter(acc_sum, [b], r)
            plsc.addupdate_scatter(acc_cnt, [b], ones16)

    with jax.named_scope("ph2_publish"):
        pltpu.sync_copy(acc_sum, sh_sum.at[w])
        pltpu.sync_copy(acc_cnt, sh_cnt.at[w])
        plsc.subcore_barrier()

    with jax.named_scope("ph2b_reduce"):
      @pl.when(w < 8)
      def _():
        seg0 = w * SEG_BLK
        pltpu.sync_copy(sh_sum.at[:, pl.ds(seg0, SEG_BLK)], red_sum)
        pltpu.sync_copy(sh_cnt.at[:, pl.ds(seg0, SEG_BLK)], red_cnt)
        for sub in range(SEG_BLK // L):
            s = jnp.zeros((L,), jnp.float32)
            c = jnp.zeros((L,), jnp.float32)
            for t in range(NTILES):
                s = s + red_sum[t, pl.ds(sub * L, L)]
                c = c + red_cnt[t, pl.ds(sub * L, L)]
            inv_v[pl.ds(sub * L, L)] = jnp.maximum(c, 1.0) / s
        pltpu.sync_copy(inv_v.at[pl.ds(0, SEG_BLK)],
                        sh_inv.at[pl.ds(seg0, SEG_BLK)])

    with jax.named_scope("ph3_bcast"):
        plsc.subcore_barrier()
        pltpu.sync_copy(sh_inv, inv_v)

    with jax.named_scope("ph4_pass2"):
        @plsc.parallel_loop(0, nelem, step=L, unroll=UNROLL)
        def _(i):
            b = bid_v[pl.ds(i, L)]
            r = raw_v[pl.ds(i, L)]
            m = plsc.load_gather(inv_v, [b])
            raw_v[pl.ds(i, L)] = r * m

    with jax.named_scope("ph5_dma_out"):
        pltpu.sync_copy(raw_v.at[pl.ds(0, CH_TAIL)],
                        out_hbm.at[pl.ds(base, CH_TAIL)])

        @pl.when(w < 15)
        def _():
            pltpu.sync_copy(raw_v.at[pl.ds(CH_TAIL, CH_MAIN - CH_TAIL)],
                            out_hbm.at[pl.ds(base + CH_TAIL, CH_MAIN - CH_TAIL)])


def kernel(atomic_numbers, batch_idx, freq_ratios):
    an = atomic_numbers.astype(jnp.int32)
    bid = batch_idx.astype(jnp.int32)
    fr = jnp.pad(freq_ratios, (0, TBL - ATOM_TYPES))
    return _sc_multiplier(an, bid, fr)


# speedup vs baseline: 1.4461x; 1.4461x over previous
import functools

import jax
import jax.numpy as jnp
from jax import lax
from jax.experimental import pallas as pl
from jax.experimental.pallas import tpu as pltpu
from jax.experimental.pallas import tpu_sc as plsc

N = 100000
ATOM_TYPES = 119
NUM_GRAPHS = 1024

L = 16
NTILES = 16
CH_MAIN = 6256
CH_TAIL = N - 15 * CH_MAIN
TBL = 128
SEG_PAD = NUM_GRAPHS
SEG_BLK = 128
UNROLL = 4

_mesh = plsc.VectorSubcoreMesh(
    core_axis_name="c", subcore_axis_name="s", num_cores=1)


@functools.partial(
    pl.kernel,
    out_type=jax.ShapeDtypeStruct((N,), jnp.float32),
    mesh=_mesh,
    compiler_params=pltpu.CompilerParams(needs_layout_passes=False),
    scratch_types=[
        pltpu.VMEM((CH_MAIN,), jnp.int32),
        pltpu.VMEM((CH_MAIN + L,), jnp.int32),
        pltpu.VMEM((CH_MAIN,), jnp.float32),
        pltpu.VMEM((TBL,), jnp.float32),
        pltpu.VMEM((SEG_PAD,), jnp.float32),
        pltpu.VMEM((SEG_PAD,), jnp.float32),
        pltpu.VMEM((NTILES, SEG_BLK), jnp.float32),
        pltpu.VMEM((NTILES, SEG_BLK), jnp.float32),
        pltpu.VMEM((SEG_PAD,), jnp.float32),
        pltpu.VMEM_SHARED((NTILES, SEG_PAD), jnp.float32),
        pltpu.VMEM_SHARED((NTILES, SEG_PAD), jnp.float32),
        pltpu.VMEM_SHARED((SEG_PAD,), jnp.float32),
        pltpu.SemaphoreType.DMA,
        pltpu.SemaphoreType.DMA,
    ],
)
def _sc_multiplier(an_hbm, bid_hbm, fr_hbm, out_hbm,
                   an_v, bid_v, raw_v, tbl_v, acc_sum, acc_cnt,
                   red_sum, red_cnt, inv_v, sh_sum, sh_cnt, sh_inv,
                   sem_a, sem_b):
    w = lax.axis_index("s")
    base = w * CH_MAIN
    nelem = jnp.where(w < 15, CH_MAIN, CH_TAIL)

    with jax.named_scope("ph0_dma_in"):
        cp_a = pltpu.async_copy(an_hbm.at[pl.ds(base, CH_TAIL)],
                                an_v.at[pl.ds(0, CH_TAIL)], sem_a)
        cp_b = pltpu.async_copy(bid_hbm.at[pl.ds(base, CH_TAIL)],
                                bid_v.at[pl.ds(0, CH_TAIL)], sem_b)
        pltpu.sync_copy(fr_hbm, tbl_v)

        eps = jnp.float32(1e-8)
        for j in range(TBL // L):
            f = tbl_v[pl.ds(j * L, L)]
            tbl_v[pl.ds(j * L, L)] = 1.0 / (f + eps)

        zero16 = jnp.zeros((L,), jnp.float32)
        for j in range(SEG_PAD // L):
            acc_sum[pl.ds(j * L, L)] = zero16
            acc_cnt[pl.ds(j * L, L)] = zero16

        @pl.when(w < 15)
        def _():
            pltpu.sync_copy(an_hbm.at[pl.ds(base + CH_TAIL, CH_MAIN - CH_TAIL)],
                            an_v.at[pl.ds(CH_TAIL, CH_MAIN - CH_TAIL)])
            pltpu.sync_copy(bid_hbm.at[pl.ds(base + CH_TAIL, CH_MAIN - CH_TAIL)],
                            bid_v.at[pl.ds(CH_TAIL, CH_MAIN - CH_TAIL)])

        cp_a.wait()
        cp_b.wait()
        bid_v[pl.ds(nelem, L)] = jnp.full((L,), -1, jnp.int32)

    iota1f = (lax.iota(jnp.int32, L) + 1).astype(jnp.float32)
    neg_iota1f = -iota1f
    lt15 = lax.iota(jnp.int32, L) < (L - 1)
    is15 = lax.iota(jnp.int32, L) == (L - 1)

    with jax.named_scope("ph1_pass1"):
        @plsc.parallel_loop(0, nelem, step=L, unroll=UNROLL)
        def _(i):
            a = an_v[pl.ds(i, L)]
            b = bid_v[pl.ds(i, L)]
            bn = bid_v[pl.ds(i + 1, L)]
            r = plsc.load_gather(tbl_v, [a])
            raw_v[pl.ds(i, L)] = r
            c = plsc.cumsum(r)
            neq = b != bn
            m_end = neq | is15
            m_start = neq & lt15
            plsc.addupdate_scatter(acc_sum, [b], c, mask=m_end)
            plsc.addupdate_scatter(acc_sum, [bn], -c, mask=m_start)
            plsc.addupdate_scatter(acc_cnt, [b], iota1f, mask=m_end)
            plsc.addupdate_scatter(acc_cnt, [bn], neg_iota1f, mask=m_start)

    with jax.named_scope("ph2_publish"):
        pltpu.sync_copy(acc_sum, sh_sum.at[w])
        pltpu.sync_copy(acc_cnt, sh_cnt.at[w])
        plsc.subcore_barrier()

    with jax.named_scope("ph2b_reduce"):
      @pl.when(w < 8)
      def _():
        seg0 = w * SEG_BLK
        pltpu.sync_copy(sh_sum.at[:, pl.ds(seg0, SEG_BLK)], red_sum)
        pltpu.sync_copy(sh_cnt.at[:, pl.ds(seg0, SEG_BLK)], red_cnt)
        for sub in range(SEG_BLK // L):
            s = jnp.zeros((L,), jnp.float32)
            c = jnp.zeros((L,), jnp.float32)
            for t in range(NTILES):
                s = s + red_sum[t, pl.ds(sub * L, L)]
                c = c + red_cnt[t, pl.ds(sub * L, L)]
            inv_v[pl.ds(sub * L, L)] = jnp.maximum(c, 1.0) / s
        pltpu.sync_copy(inv_v.at[pl.ds(0, SEG_BLK)],
                        sh_inv.at[pl.ds(seg0, SEG_BLK)])

    with jax.named_scope("ph3_bcast"):
        plsc.subcore_barrier()
        pltpu.sync_copy(sh_inv, inv_v)

    with jax.named_scope("ph4_pass2"):
        @plsc.parallel_loop(0, nelem, step=L, unroll=UNROLL)
        def _(i):
            b = bid_v[pl.ds(i, L)]
            r = raw_v[pl.ds(i, L)]
            m = plsc.load_gather(inv_v, [b])
            raw_v[pl.ds(i, L)] = r * m

    with jax.named_scope("ph5_dma_out"):
        pltpu.sync_copy(raw_v.at[pl.ds(0, CH_TAIL)],
                        out_hbm.at[pl.ds(base, CH_TAIL)])

        @pl.when(w < 15)
        def _():
            pltpu.sync_copy(raw_v.at[pl.ds(CH_TAIL, CH_MAIN - CH_TAIL)],
                            out_hbm.at[pl.ds(base + CH_TAIL, CH_MAIN - CH_TAIL)])


def kernel(atomic_numbers, batch_idx, freq_ratios):
    an = atomic_numbers.astype(jnp.int32)
    bid = batch_idx.astype(jnp.int32)
    fr = jnp.pad(freq_ratios, (0, TBL - ATOM_TYPES))
    return _sc_multiplier(an, bid, fr)


# speedup vs baseline: 1.4638x; 1.0122x over previous
import functools

import jax
import jax.numpy as jnp
from jax import lax
from jax.experimental import pallas as pl
from jax.experimental.pallas import tpu as pltpu
from jax.experimental.pallas import tpu_sc as plsc

N = 100000
ATOM_TYPES = 119
NUM_GRAPHS = 1024

L = 16
NTILES = 16
CH_MAIN = 6256
CH_TAIL = N - 15 * CH_MAIN
CH_HALF = 3136
TBL = 128
ACC = 2 * NUM_GRAPHS
SEG_BLK = 128
UNROLL = 4

_mesh = plsc.VectorSubcoreMesh(
    core_axis_name="c", subcore_axis_name="s", num_cores=1)


@functools.partial(
    pl.kernel,
    out_type=jax.ShapeDtypeStruct((N,), jnp.float32),
    mesh=_mesh,
    compiler_params=pltpu.CompilerParams(needs_layout_passes=False),
    scratch_types=[
        pltpu.VMEM((CH_MAIN,), jnp.int32),
        pltpu.VMEM((CH_MAIN + L,), jnp.int32),
        pltpu.VMEM((CH_MAIN,), jnp.float32),
        pltpu.VMEM((TBL,), jnp.float32),
        pltpu.VMEM((ACC,), jnp.float32),
        pltpu.VMEM((NTILES, SEG_BLK), jnp.float32),
        pltpu.VMEM((NTILES, SEG_BLK), jnp.float32),
        pltpu.VMEM((NUM_GRAPHS,), jnp.float32),
        pltpu.VMEM_SHARED((NTILES, ACC), jnp.float32),
        pltpu.VMEM_SHARED((NUM_GRAPHS,), jnp.float32),
        pltpu.SemaphoreType.DMA,
        pltpu.SemaphoreType.DMA,
    ],
)
def _sc_multiplier(an_hbm, bid_hbm, fr_hbm, out_hbm,
                   an_v, bid_v, raw_v, tbl_v, acc,
                   red_sum, red_cnt, inv_v, sh_acc, sh_inv,
                   sem_a, sem_b):
    w = lax.axis_index("s")
    base = w * CH_MAIN
    nelem = jnp.where(w < 15, CH_MAIN, CH_TAIL)

    cp_a = pltpu.async_copy(an_hbm.at[pl.ds(base, CH_TAIL)],
                            an_v.at[pl.ds(0, CH_TAIL)], sem_a)
    cp_b = pltpu.async_copy(bid_hbm.at[pl.ds(base, CH_TAIL)],
                            bid_v.at[pl.ds(0, CH_TAIL)], sem_b)
    pltpu.sync_copy(fr_hbm, tbl_v.at[pl.ds(0, ATOM_TYPES)])

    eps = jnp.float32(1e-8)
    for j in range(TBL // L):
        f = tbl_v[pl.ds(j * L, L)]
        tbl_v[pl.ds(j * L, L)] = 1.0 / (f + eps)

    zero16 = jnp.zeros((L,), jnp.float32)
    for j in range(ACC // L):
        acc[pl.ds(j * L, L)] = zero16

    @pl.when(w < 15)
    def _():
        pltpu.sync_copy(an_hbm.at[pl.ds(base + CH_TAIL, CH_MAIN - CH_TAIL)],
                        an_v.at[pl.ds(CH_TAIL, CH_MAIN - CH_TAIL)])
        pltpu.sync_copy(bid_hbm.at[pl.ds(base + CH_TAIL, CH_MAIN - CH_TAIL)],
                        bid_v.at[pl.ds(CH_TAIL, CH_MAIN - CH_TAIL)])

    cp_a.wait()
    cp_b.wait()
    bid_v[pl.ds(nelem, L)] = jnp.full((L,), -1, jnp.int32)

    iota1f = (lax.iota(jnp.int32, L) + 1).astype(jnp.float32)
    neg_iota1f = -iota1f
    lt15 = lax.iota(jnp.int32, L) < (L - 1)
    is15 = lax.iota(jnp.int32, L) == (L - 1)
    cnt_off = jnp.full((L,), NUM_GRAPHS, jnp.int32)

    @plsc.parallel_loop(0, nelem, step=L, unroll=UNROLL)
    def _(i):
        a = an_v[pl.ds(i, L)]
        b = bid_v[pl.ds(i, L)]
        bn = bid_v[pl.ds(i + 1, L)]
        r = plsc.load_gather(tbl_v, [a])
        raw_v[pl.ds(i, L)] = r
        c = plsc.cumsum(r)
        neq = b != bn
        m_end = neq | is15
        m_start = neq & lt15
        plsc.addupdate_scatter(acc, [b], c, mask=m_end)
        plsc.addupdate_scatter(acc, [bn], -c, mask=m_start)
        plsc.addupdate_scatter(acc, [b + cnt_off], iota1f, mask=m_end)
        plsc.addupdate_scatter(acc, [bn + cnt_off], neg_iota1f, mask=m_start)

    pltpu.sync_copy(acc, sh_acc.at[w])
    plsc.subcore_barrier()

    @pl.when(w < 8)
    def _():
        seg0 = w * SEG_BLK
        pltpu.sync_copy(sh_acc.at[:, pl.ds(seg0, SEG_BLK)], red_sum)
        pltpu.sync_copy(sh_acc.at[:, pl.ds(NUM_GRAPHS + seg0, SEG_BLK)],
                        red_cnt)
        for sub in range(SEG_BLK // L):
            s = jnp.zeros((L,), jnp.float32)
            c = jnp.zeros((L,), jnp.float32)
            for t in range(NTILES):
                s = s + red_sum[t, pl.ds(sub * L, L)]
                c = c + red_cnt[t, pl.ds(sub * L, L)]
            inv_v[pl.ds(sub * L, L)] = jnp.maximum(c, 1.0) / s
        pltpu.sync_copy(inv_v.at[pl.ds(0, SEG_BLK)],
                        sh_inv.at[pl.ds(seg0, SEG_BLK)])

    plsc.subcore_barrier()
    pltpu.sync_copy(sh_inv, inv_v)

    @plsc.parallel_loop(0, CH_HALF, step=L, unroll=UNROLL)
    def _(i):
        b = bid_v[pl.ds(i, L)]
        r = raw_v[pl.ds(i, L)]
        m = plsc.load_gather(inv_v, [b])
        raw_v[pl.ds(i, L)] = r * m

    cp_out = pltpu.async_copy(raw_v.at[pl.ds(0, CH_HALF)],
                              out_hbm.at[pl.ds(base, CH_HALF)], sem_a)

    @plsc.parallel_loop(CH_HALF, nelem, step=L, unroll=UNROLL)
    def _(i):
        b = bid_v[pl.ds(i, L)]
        r = raw_v[pl.ds(i, L)]
        m = plsc.load_gather(inv_v, [b])
        raw_v[pl.ds(i, L)] = r * m

    pltpu.sync_copy(raw_v.at[pl.ds(CH_HALF, CH_TAIL - CH_HALF)],
                    out_hbm.at[pl.ds(base + CH_HALF, CH_TAIL - CH_HALF)])

    @pl.when(w < 15)
    def _():
        pltpu.sync_copy(raw_v.at[pl.ds(CH_TAIL, CH_MAIN - CH_TAIL)],
                        out_hbm.at[pl.ds(base + CH_TAIL, CH_MAIN - CH_TAIL)])

    cp_out.wait()


def kernel(atomic_numbers, batch_idx, freq_ratios):
    return _sc_multiplier(atomic_numbers.astype(jnp.int32),
                          batch_idx.astype(jnp.int32),
                          freq_ratios)


# speedup vs baseline: 1.5096x; 1.0313x over previous
import functools

import jax
import jax.numpy as jnp
from jax import lax
from jax.experimental import pallas as pl
from jax.experimental.pallas import tpu as pltpu
from jax.experimental.pallas import tpu_sc as plsc

N = 100000
ATOM_TYPES = 119
NUM_GRAPHS = 1024

L = 16
NTILES = 16
CH_MAIN = 6256
CH_TAIL = N - 15 * CH_MAIN
CH_HALF = 3136
TBL = 128
ACC = 2 * NUM_GRAPHS
SEG_BLK = 128
UNROLL = 4

_mesh = plsc.VectorSubcoreMesh(
    core_axis_name="c", subcore_axis_name="s", num_cores=1)


@functools.partial(
    pl.kernel,
    out_type=jax.ShapeDtypeStruct((N,), jnp.float32),
    mesh=_mesh,
    compiler_params=pltpu.CompilerParams(needs_layout_passes=False),
    scratch_types=[
        pltpu.VMEM((CH_MAIN,), jnp.int32),
        pltpu.VMEM((CH_MAIN + L,), jnp.int32),
        pltpu.VMEM((CH_MAIN,), jnp.float32),
        pltpu.VMEM((TBL,), jnp.float32),
        pltpu.VMEM((ACC,), jnp.float32),
        pltpu.VMEM((NTILES, SEG_BLK), jnp.float32),
        pltpu.VMEM((NTILES, SEG_BLK), jnp.float32),
        pltpu.VMEM((NUM_GRAPHS,), jnp.float32),
        pltpu.VMEM_SHARED((NTILES, ACC), jnp.float32),
        pltpu.VMEM_SHARED((NUM_GRAPHS,), jnp.float32),
        pltpu.SemaphoreType.DMA,
        pltpu.SemaphoreType.DMA,
        pltpu.SemaphoreType.DMA,
    ],
)
def _sc_multiplier(an_hbm, bid_hbm, fr_hbm, out_hbm,
                   an_v, bid_v, raw_v, tbl_v, acc,
                   red_sum, red_cnt, inv_v, sh_acc, sh_inv,
                   sem_a, sem_b, sem_c):
    w = lax.axis_index("s")
    base = w * CH_MAIN
    nelem = jnp.where(w < 15, CH_MAIN, CH_TAIL)
    tail_off = jnp.where(w < 15, base + CH_TAIL, 0)

    cp_a = pltpu.async_copy(an_hbm.at[pl.ds(base, CH_TAIL)],
                            an_v.at[pl.ds(0, CH_TAIL)], sem_a)
    cp_b = pltpu.async_copy(bid_hbm.at[pl.ds(base, CH_TAIL)],
                            bid_v.at[pl.ds(0, CH_TAIL)], sem_b)
    cp_c = pltpu.async_copy(an_hbm.at[pl.ds(tail_off, CH_MAIN - CH_TAIL)],
                            an_v.at[pl.ds(CH_TAIL, CH_MAIN - CH_TAIL)], sem_a)
    cp_d = pltpu.async_copy(bid_hbm.at[pl.ds(tail_off, CH_MAIN - CH_TAIL)],
                            bid_v.at[pl.ds(CH_TAIL, CH_MAIN - CH_TAIL)], sem_b)
    cp_f = pltpu.async_copy(fr_hbm, tbl_v.at[pl.ds(0, ATOM_TYPES)], sem_c)

    zero16 = jnp.zeros((L,), jnp.float32)
    for j in range(ACC // L):
        acc[pl.ds(j * L, L)] = zero16

    cp_f.wait()
    eps = jnp.float32(1e-8)
    for j in range(TBL // L):
        f = tbl_v[pl.ds(j * L, L)]
        tbl_v[pl.ds(j * L, L)] = 1.0 / (f + eps)

    cp_a.wait()
    cp_c.wait()
    cp_b.wait()
    cp_d.wait()
    bid_v[pl.ds(nelem, L)] = jnp.full((L,), -1, jnp.int32)

    iota1f = (lax.iota(jnp.int32, L) + 1).astype(jnp.float32)
    neg_iota1f = -iota1f
    lt15 = lax.iota(jnp.int32, L) < (L - 1)
    is15 = lax.iota(jnp.int32, L) == (L - 1)
    cnt_off = jnp.full((L,), NUM_GRAPHS, jnp.int32)

    @plsc.parallel_loop(0, nelem, step=L, unroll=UNROLL)
    def _(i):
        a = an_v[pl.ds(i, L)]
        b = bid_v[pl.ds(i, L)]
        bn = bid_v[pl.ds(i + 1, L)]
        r = plsc.load_gather(tbl_v, [a])
        raw_v[pl.ds(i, L)] = r
        c = plsc.cumsum(r)
        neq = b != bn
        m_end = neq | is15
        m_start = neq & lt15
        plsc.addupdate_scatter(acc, [b], c, mask=m_end)
        plsc.addupdate_scatter(acc, [bn], -c, mask=m_start)
        plsc.addupdate_scatter(acc, [b + cnt_off], iota1f, mask=m_end)
        plsc.addupdate_scatter(acc, [bn + cnt_off], neg_iota1f, mask=m_start)

    pltpu.sync_copy(acc, sh_acc.at[w])
    plsc.subcore_barrier()

    @pl.when(w < 8)
    def _():
        seg0 = w * SEG_BLK
        pltpu.sync_copy(sh_acc.at[:, pl.ds(seg0, SEG_BLK)], red_sum)
        pltpu.sync_copy(sh_acc.at[:, pl.ds(NUM_GRAPHS + seg0, SEG_BLK)],
                        red_cnt)
        for sub in range(SEG_BLK // L):
            s = jnp.zeros((L,), jnp.float32)
            c = jnp.zeros((L,), jnp.float32)
            for t in range(NTILES):
                s = s + red_sum[t, pl.ds(sub * L, L)]
                c = c + red_cnt[t, pl.ds(sub * L, L)]
            inv_v[pl.ds(sub * L, L)] = jnp.maximum(c, 1.0) / s
        pltpu.sync_copy(inv_v.at[pl.ds(0, SEG_BLK)],
                        sh_inv.at[pl.ds(seg0, SEG_BLK)])

    plsc.subcore_barrier()
    pltpu.sync_copy(sh_inv, inv_v)

    @plsc.parallel_loop(0, CH_HALF, step=L, unroll=UNROLL)
    def _(i):
        b = bid_v[pl.ds(i, L)]
        r = raw_v[pl.ds(i, L)]
        m = plsc.load_gather(inv_v, [b])
        raw_v[pl.ds(i, L)] = r * m

    cp_out = pltpu.async_copy(raw_v.at[pl.ds(0, CH_HALF)],
                              out_hbm.at[pl.ds(base, CH_HALF)], sem_a)

    @plsc.parallel_loop(CH_HALF, nelem, step=L, unroll=UNROLL)
    def _(i):
        b = bid_v[pl.ds(i, L)]
        r = raw_v[pl.ds(i, L)]
        m = plsc.load_gather(inv_v, [b])
        raw_v[pl.ds(i, L)] = r * m

    @pl.when(w < 15)
    def _():
        pltpu.sync_copy(raw_v.at[pl.ds(CH_HALF, CH_MAIN - CH_HALF)],
                        out_hbm.at[pl.ds(base + CH_HALF, CH_MAIN - CH_HALF)])

    @pl.when(w == 15)
    def _():
        pltpu.sync_copy(raw_v.at[pl.ds(CH_HALF, CH_TAIL - CH_HALF)],
                        out_hbm.at[pl.ds(base + CH_HALF, CH_TAIL - CH_HALF)])

    cp_out.wait()


def kernel(atomic_numbers, batch_idx, freq_ratios):
    return _sc_multiplier(atomic_numbers.astype(jnp.int32),
                          batch_idx.astype(jnp.int32),
                          freq_ratios)
